# baseline (speedup 1.0000x reference)
import jax
import jax.numpy as jnp
from jax.experimental import pallas as pl
from jax.experimental.pallas import tpu as pltpu

_NDIM = 64
_HDIM = 256
_ODIM = 64
_WINDOW = 32
_MEM = 16
_CTX = 64


def _gru_step(gi, h, whh, bhh_n, hdim):
    gh = jnp.dot(h.astype(jnp.bfloat16), whh,
                 preferred_element_type=jnp.float32)
    r = jax.nn.sigmoid(gi[:, :hdim] + gh[:, :hdim])
    z = jax.nn.sigmoid(gi[:, hdim:2 * hdim] + gh[:, hdim:2 * hdim])
    n = jnp.tanh(gi[:, 2 * hdim:] + r * (gh[:, 2 * hdim:] + bhh_n))
    return n + z * (h - n)


def _fused_kernel(seq_ref,
                  wih_in_ref, whh_in_ref, bih_in_ref, bhh_in_ref,
                  wih_mem_ref, whh_mem_ref, bih_mem_ref, bhh_mem_ref,
                  wih_ctx_ref, whh_ctx_ref, bih_ctx_ref, bhh_ctx_ref,
                  wlin_emb_ref, wlin_ctx_ref, blin_ref,
                  out_ref,
                  gi_mem, gi_ctx, gi_in, h_mem, h_ctx, h_in):
    B = out_ref.shape[0]
    W = _WINDOW
    M = _MEM
    seq = seq_ref[...]

    f32 = jnp.float32
    gi_in[...] = (jnp.dot(seq, wih_in_ref[...], preferred_element_type=f32)
                  + bih_in_ref[...]).astype(jnp.bfloat16)
    gm = ((jnp.dot(seq, wih_mem_ref[...], preferred_element_type=f32)
          + bih_mem_ref[...])).astype(jnp.bfloat16)
    gi_mem[:W * B, :] = gm
    gi_mem[W * B:, :] = gm[:M * B, :]
    gc = (jnp.dot(seq, wih_ctx_ref[...], preferred_element_type=f32)
          + bih_ctx_ref[...]).astype(jnp.bfloat16)
    gi_ctx[:W * B, :] = gc
    gi_ctx[W * B:, :] = gc[:M * B, :]

    h_mem[...] = jnp.zeros_like(h_mem)
    h_ctx[...] = jnp.zeros_like(h_ctx)
    h_in[...] = jnp.zeros_like(h_in)

    def step(t, carry):
        h_mem[...] = _gru_step(gi_mem[pl.ds(t * B, M * B), :], h_mem[...],
                               whh_mem_ref[...], bhh_mem_ref[...], _HDIM)
        h_ctx[...] = _gru_step(gi_ctx[pl.ds(t * B, M * B), :], h_ctx[...],
                               whh_ctx_ref[...], bhh_ctx_ref[...], _CTX)
        h_in[...] = _gru_step(gi_in[pl.ds(t * B, B), :], h_in[...],
                              whh_in_ref[...], bhh_in_ref[...], _HDIM)
        return carry

    jax.lax.fori_loop(0, W, step, 0, unroll=8)

    emb = h_in[...]
    mem3 = h_mem[...].reshape(M, B, _HDIM)
    scores = jnp.sum(mem3 * emb[None, :, :], axis=-1)
    smax = jnp.max(scores, axis=0, keepdims=True)
    e = jnp.exp(scores - smax)
    attn = e / jnp.sum(e, axis=0, keepdims=True)
    ctx3 = h_ctx[...].reshape(M, B, _CTX)
    retrieved = jnp.sum(ctx3 * attn[:, :, None], axis=0)

    out_ref[...] = (jnp.dot(emb, wlin_emb_ref[...], preferred_element_type=f32)
                    + jnp.dot(retrieved, wlin_ctx_ref[...],
                              preferred_element_type=f32)
                    + blin_ref[...])


def kernel(x, gru_input_W_ih, gru_input_W_hh, gru_input_b_ih, gru_input_b_hh,
           gru_memory_W_ih, gru_memory_W_hh, gru_memory_b_ih, gru_memory_b_hh,
           gru_context_W_ih, gru_context_W_hh, gru_context_b_ih,
           gru_context_b_hh, W_lin, b_lin):
    B = x.shape[0]
    W = _WINDOW
    M = _MEM
    seq = jnp.transpose(x.reshape(B, _NDIM, W), (2, 0, 1)).reshape(W * B, _NDIM)

    r2 = lambda v: v.reshape(1, -1)

    def _fold(b_ih, b_hh, hdim):
        folded = b_ih.at[:2 * hdim].add(b_hh[:2 * hdim])
        return r2(folded), r2(b_hh[2 * hdim:])

    bi_in, bn_in = _fold(gru_input_b_ih, gru_input_b_hh, _HDIM)
    bi_mem, bn_mem = _fold(gru_memory_b_ih, gru_memory_b_hh, _HDIM)
    bi_ctx, bn_ctx = _fold(gru_context_b_ih, gru_context_b_hh, _CTX)
    args = (
        seq,
        gru_input_W_ih.T, gru_input_W_hh.T.astype(jnp.bfloat16), bi_in, bn_in,
        gru_memory_W_ih.T, gru_memory_W_hh.T.astype(jnp.bfloat16), bi_mem, bn_mem,
        gru_context_W_ih.T, gru_context_W_hh.T.astype(jnp.bfloat16), bi_ctx, bn_ctx,
        W_lin[:, :_HDIM].T, W_lin[:, _HDIM:].T, r2(b_lin),
    )

    out = pl.pallas_call(
        _fused_kernel,
        out_shape=jax.ShapeDtypeStruct((B, _ODIM), jnp.float32),
        scratch_shapes=[
            pltpu.VMEM(((W + M) * B, 3 * _HDIM), jnp.bfloat16),
            pltpu.VMEM(((W + M) * B, 3 * _CTX), jnp.bfloat16),
            pltpu.VMEM((W * B, 3 * _HDIM), jnp.bfloat16),
            pltpu.VMEM((M * B, _HDIM), jnp.float32),
            pltpu.VMEM((M * B, _CTX), jnp.float32),
            pltpu.VMEM((B, _HDIM), jnp.float32),
        ],
    )(*args)
    return out.reshape(B, _ODIM, 1, 1, 1)


# speedup vs baseline: 1.0278x; 1.0278x over previous
import jax
import jax.numpy as jnp
from jax.experimental import pallas as pl
from jax.experimental.pallas import tpu as pltpu

_NDIM = 64
_HDIM = 256
_ODIM = 64
_WINDOW = 32
_MEM = 16
_CTX = 64


def _gru_step(gi, h, whh, bhh_n, hdim):
    gh = jnp.dot(h.astype(jnp.bfloat16), whh,
                 preferred_element_type=jnp.float32)
    r = jax.nn.sigmoid(gi[:, :hdim] + gh[:, :hdim])
    z = jax.nn.sigmoid(gi[:, hdim:2 * hdim] + gh[:, hdim:2 * hdim])
    n = jnp.tanh(gi[:, 2 * hdim:] + r * (gh[:, 2 * hdim:] + bhh_n))
    return n + z * (h - n)


def _fused_kernel(seq_ref,
                  wih_in_ref, whh_in_ref, bih_in_ref, bhh_in_ref,
                  wih_mem_ref, whh_mem_ref, bih_mem_ref, bhh_mem_ref,
                  wih_ctx_ref, whh_ctx_ref, bih_ctx_ref, bhh_ctx_ref,
                  wlin_emb_ref, wlin_ctx_ref, blin_ref,
                  out_ref,
                  gi_mem, gi_ctx, gi_in, h_mem, h_ctx, h_in):
    B = out_ref.shape[0]
    W = _WINDOW
    M = _MEM
    seq = seq_ref[...]

    f32 = jnp.float32
    gi_in[...] = (jnp.dot(seq, wih_in_ref[...], preferred_element_type=f32)
                  + bih_in_ref[...])
    gm = (jnp.dot(seq, wih_mem_ref[...], preferred_element_type=f32)
          + bih_mem_ref[...])
    gi_mem[:W * B, :] = gm
    gi_mem[W * B:, :] = gm[:M * B, :]
    gc = (jnp.dot(seq, wih_ctx_ref[...], preferred_element_type=f32)
          + bih_ctx_ref[...])
    gi_ctx[:W * B, :] = gc
    gi_ctx[W * B:, :] = gc[:M * B, :]

    h_mem[...] = jnp.zeros_like(h_mem)
    h_ctx[...] = jnp.zeros_like(h_ctx)
    h_in[...] = jnp.zeros_like(h_in)

    def step(t, carry):
        h_mem[...] = _gru_step(gi_mem[pl.ds(t * B, M * B), :], h_mem[...],
                               whh_mem_ref[...], bhh_mem_ref[...], _HDIM)
        h_ctx[...] = _gru_step(gi_ctx[pl.ds(t * B, M * B), :], h_ctx[...],
                               whh_ctx_ref[...], bhh_ctx_ref[...], _CTX)
        h_in[...] = _gru_step(gi_in[pl.ds(t * B, B), :], h_in[...],
                              whh_in_ref[...], bhh_in_ref[...], _HDIM)
        return carry

    jax.lax.fori_loop(0, W, step, 0, unroll=8)

    emb = h_in[...]
    mem3 = h_mem[...].reshape(M, B, _HDIM)
    scores = jnp.sum(mem3 * emb[None, :, :], axis=-1)
    smax = jnp.max(scores, axis=0, keepdims=True)
    e = jnp.exp(scores - smax)
    attn = e / jnp.sum(e, axis=0, keepdims=True)
    ctx3 = h_ctx[...].reshape(M, B, _CTX)
    retrieved = jnp.sum(ctx3 * attn[:, :, None], axis=0)

    out_ref[...] = (jnp.dot(emb, wlin_emb_ref[...], preferred_element_type=f32)
                    + jnp.dot(retrieved, wlin_ctx_ref[...],
                              preferred_element_type=f32)
                    + blin_ref[...])


def kernel(x, gru_input_W_ih, gru_input_W_hh, gru_input_b_ih, gru_input_b_hh,
           gru_memory_W_ih, gru_memory_W_hh, gru_memory_b_ih, gru_memory_b_hh,
           gru_context_W_ih, gru_context_W_hh, gru_context_b_ih,
           gru_context_b_hh, W_lin, b_lin):
    B = x.shape[0]
    W = _WINDOW
    M = _MEM
    seq = jnp.transpose(x.reshape(B, _NDIM, W), (2, 0, 1)).reshape(W * B, _NDIM)

    r2 = lambda v: v.reshape(1, -1)

    def _fold(b_ih, b_hh, hdim):
        folded = b_ih.at[:2 * hdim].add(b_hh[:2 * hdim])
        return r2(folded), r2(b_hh[2 * hdim:])

    bi_in, bn_in = _fold(gru_input_b_ih, gru_input_b_hh, _HDIM)
    bi_mem, bn_mem = _fold(gru_memory_b_ih, gru_memory_b_hh, _HDIM)
    bi_ctx, bn_ctx = _fold(gru_context_b_ih, gru_context_b_hh, _CTX)
    args = (
        seq,
        gru_input_W_ih.T, gru_input_W_hh.T.astype(jnp.bfloat16), bi_in, bn_in,
        gru_memory_W_ih.T, gru_memory_W_hh.T.astype(jnp.bfloat16), bi_mem, bn_mem,
        gru_context_W_ih.T, gru_context_W_hh.T.astype(jnp.bfloat16), bi_ctx, bn_ctx,
        W_lin[:, :_HDIM].T, W_lin[:, _HDIM:].T, r2(b_lin),
    )

    out = pl.pallas_call(
        _fused_kernel,
        out_shape=jax.ShapeDtypeStruct((B, _ODIM), jnp.float32),
        scratch_shapes=[
            pltpu.VMEM(((W + M) * B, 3 * _HDIM), jnp.float32),
            pltpu.VMEM(((W + M) * B, 3 * _CTX), jnp.float32),
            pltpu.VMEM((W * B, 3 * _HDIM), jnp.float32),
            pltpu.VMEM((M * B, _HDIM), jnp.float32),
            pltpu.VMEM((M * B, _CTX), jnp.float32),
            pltpu.VMEM((B, _HDIM), jnp.float32),
        ],
    )(*args)
    return out.reshape(B, _ODIM, 1, 1, 1)
